# baseline (speedup 1.0000x reference)
import functools

import jax
import jax.numpy as jnp
from jax import lax
from jax.experimental import pallas as pl
from jax.experimental.pallas import tpu as pltpu
from jax.experimental.pallas import tpu_sc as plsc

NUM_CORES = 2
NUM_SUBCORES = 16
NUM_WORKERS = NUM_CORES * NUM_SUBCORES

STREAM = 128
STREAMS_PER_SUPER = 2
SUPER = STREAM * STREAMS_PER_SUPER


def _gather_body(per_w, n_super, d, table_hbm, idx_hbm, out_hbm,
                 idx_v, rows0, rows1, sem0, sem1):
    wid = lax.axis_index("s") * NUM_CORES + lax.axis_index("c")
    base = wid * per_w
    pltpu.sync_copy(idx_hbm.at[wid], idx_v)

    bufs = (rows0, rows1)
    sems = (sem0, sem1)

    def issue(sup):
        buf = bufs[sup % 2]
        sem = sems[sup % 2]
        cps = []
        for j in range(STREAMS_PER_SUPER):
            s = sup * STREAMS_PER_SUPER + j
            cps.append(pltpu.async_copy(
                table_hbm.at[idx_v.at[s]],
                buf.at[pl.ds(j * STREAM, STREAM)],
                sem))
        return cps

    pending = [issue(0), None]
    for sup in range(n_super):
        nxt = sup + 1
        if nxt < n_super:
            pending[nxt % 2] = issue(nxt)
        for cp in pending[sup % 2]:
            cp.wait()
        pltpu.sync_copy(bufs[sup % 2],
                        out_hbm.at[pl.ds(base + sup * SUPER, SUPER)])


def _sc_gather(table4, idx4):
    n = idx4.shape[0]
    d = table4.shape[1]
    per_w = n // NUM_WORKERS
    n_super = per_w // SUPER
    assert per_w % SUPER == 0
    mesh = plsc.VectorSubcoreMesh(core_axis_name="c", subcore_axis_name="s")
    f = pl.kernel(
        functools.partial(_gather_body, per_w, n_super, d),
        out_type=jax.ShapeDtypeStruct((n, d), jnp.float32),
        mesh=mesh,
        scratch_types=[
            pltpu.VMEM((per_w // STREAM, STREAM), jnp.int32),
            pltpu.VMEM((SUPER, d), jnp.float32),
            pltpu.VMEM((SUPER, d), jnp.float32),
            pltpu.SemaphoreType.DMA,
            pltpu.SemaphoreType.DMA,
        ],
    )
    return f(table4, idx4.reshape(NUM_WORKERS, per_w // STREAM, STREAM))


def _transpose_sc_body(n_full_tiles, t4_hbm, tail_hbm, out_hbm,
                       in0, in1, in2, in3, ob0, ob1, ob2, ob3,
                       sem0, sem1, sem2, sem3,
                       osem0, osem1, osem2, osem3):
    wid = lax.axis_index("s") * NUM_CORES + lax.axis_index("c")
    n_iter = (n_full_tiles + NUM_WORKERS - 1) // NUM_WORKERS
    n_iter += (-n_iter) % 4

    ins = (in0, in1, in2, in3)
    obs = (ob0, ob1, ob2, ob3)
    sems = (sem0, sem1, sem2, sem3)
    osems = (osem0, osem1, osem2, osem3)
    NB = 4

    lane = lax.iota(jnp.int32, 16)

    def tile_of(k):
        return wid + k * NUM_WORKERS

    def issue_in(k, p):
        t = tile_of(k)

        @pl.when(t < n_full_tiles)
        def _():
            off = pl.multiple_of(t * 128, 128)
            pltpu.async_copy(t4_hbm.at[:, :, pl.ds(off, 128)],
                             ins[p], sems[p])

    def wait_in(k, p):
        t = tile_of(k)
        off = pl.multiple_of(t * 128, 128)
        pltpu.make_async_copy(t4_hbm.at[:, :, pl.ds(off, 128)],
                              ins[p], sems[p]).wait()

    def wait_out(k, p):
        t = tile_of(k)
        pltpu.make_async_copy(obs[p], out_hbm.at[pl.ds(32 * t, 32)],
                              osems[p]).wait()

    def reclaim(k, p):
        tp = tile_of(k - NB)

        @pl.when(jnp.logical_and(tp >= 0, tp < n_full_tiles))
        def _():
            wait_out(k - NB, p)

    def step(k, p):
        reclaim(k, p)
        t = tile_of(k)

        @pl.when(t < n_full_tiles)
        def _():
            wait_in(k, p)
            ib = ins[p]
            ob = obs[p]
            dlo = lane
            dhi = lane + 16
            for r2 in range(16):
                vals = []
                for rr in range(2):
                    r = 2 * r2 + rr
                    for v in range(8):
                        q = v >> 1
                        d = dhi if (v & 1) else dlo
                        vals.append(plsc.load_gather(
                            ib, [d >> 3, d & 7,
                                 jnp.full((16,), 4 * r + q, jnp.int32)]))
                for rr in range(2):
                    r = 2 * r2 + rr
                    for v in range(8):
                        ob[r, pl.ds(16 * v, 16)] = vals[8 * rr + v]
            pltpu.async_copy(ob, out_hbm.at[pl.ds(32 * t, 32)], osems[p])

    def body(base, _):
        for p in range(NB):
            k = NB * base + p
            issue_in(k + NB - 1, (p + NB - 1) % NB)
            step(k, p)
        return 0

    for kk in range(3):
        issue_in(kk, kk)
    lax.fori_loop(0, n_iter // NB, body, 0)
    for kk in range(NB):
        k = n_iter + kk
        reclaim(k, k % NB)

    @pl.when(wid == 0)
    def _():
        pltpu.sync_copy(tail_hbm, ob0.at[pl.ds(0, 16)])
        pltpu.sync_copy(ob0.at[pl.ds(0, 16)],
                        out_hbm.at[pl.ds(32 * n_full_tiles, 16)])


def _sc_transpose(t4, tail4, n_rows):
    n_full_tiles = (t4.shape[2] // 128)
    mesh = plsc.VectorSubcoreMesh(core_axis_name="c", subcore_axis_name="s")
    f = pl.kernel(
        functools.partial(_transpose_sc_body, n_full_tiles),
        out_type=jax.ShapeDtypeStruct((n_rows, 128), jnp.float32),
        mesh=mesh,
        scratch_types=(
            [pltpu.VMEM((4, 8, 128), jnp.float32)] * 4
            + [pltpu.VMEM((32, 128), jnp.float32)] * 4
            + [pltpu.SemaphoreType.DMA] * 8
        ),
        compiler_params=pltpu.CompilerParams(needs_layout_passes=False),
    )
    return f(t4, tail4)


def _select_matmul_body(g_ref, x_ref, w_ref, b_ref, out_ref):
    g = g_ref[0]
    xv = x_ref[0]
    pos = jnp.reshape(xv & 3, (xv.shape[1], 1))
    quarter = lax.broadcasted_iota(jnp.int32, g.shape, 1) >> 5
    masked = jnp.where(quarter == pos, g, 0.0)
    m = jnp.dot(masked, w_ref[...], preferred_element_type=jnp.float32)
    out_ref[0] = jnp.transpose(m + b_ref[...])


def _tc_select_matmul(g3, xT3, Wstack, b):
    seq, bsz, d4 = g3.shape
    o = Wstack.shape[1]
    return pl.pallas_call(
        _select_matmul_body,
        grid=(seq,),
        in_specs=[
            pl.BlockSpec((1, bsz, d4), lambda i: (i, 0, 0)),
            pl.BlockSpec((1, 1, bsz), lambda i: (i, 0, 0)),
            pl.BlockSpec((d4, o), lambda i: (0, 0)),
            pl.BlockSpec((1, o), lambda i: (0, 0)),
        ],
        out_specs=pl.BlockSpec((1, o, bsz), lambda i: (i, 0, 0)),
        out_shape=jax.ShapeDtypeStruct((seq, o, bsz), jnp.float32),
    )(g3, xT3, Wstack, b.reshape(1, o))


def kernel(x, table, W, b):
    bsz, seq = x.shape
    o = W.shape[1]
    xT = x.T.astype(jnp.int32)
    xf = xT.reshape(-1)
    n_rows = table.shape[0]
    n_full = (n_rows // 128) * 128
    t4 = table.T.reshape(4, 8, n_rows)
    tail4 = table.T[:, n_full:].T.reshape((n_rows - n_full) // 4, 128)
    table4 = _sc_transpose(t4, tail4, n_rows // 4)
    g = _sc_gather(table4, xf >> 2)
    g3 = g.reshape(seq, bsz, table4.shape[1])
    Wstack = jnp.concatenate([W, W, W, W], axis=0)
    outT = _tc_select_matmul(g3, xT.reshape(seq, 1, bsz), Wstack, b)
    return outT.transpose(2, 0, 1)


# speedup vs baseline: 1.0471x; 1.0471x over previous
import functools

import jax
import jax.numpy as jnp
from jax import lax
from jax.experimental import pallas as pl
from jax.experimental.pallas import tpu as pltpu
from jax.experimental.pallas import tpu_sc as plsc

NUM_CORES = 2
NUM_SUBCORES = 16
NUM_WORKERS = NUM_CORES * NUM_SUBCORES

STREAM = 128
STREAMS_PER_SUPER = 2
SUPER = STREAM * STREAMS_PER_SUPER


def _gather_body(per_w, n_super, d, table_hbm, idx_hbm, out_hbm,
                 idx_v, rows0, rows1, sem0, sem1):
    wid = lax.axis_index("s") * NUM_CORES + lax.axis_index("c")
    base = wid * per_w
    pltpu.sync_copy(idx_hbm.at[wid], idx_v)

    bufs = (rows0, rows1)
    sems = (sem0, sem1)

    def issue(sup):
        buf = bufs[sup % 2]
        sem = sems[sup % 2]
        cps = []
        for j in range(STREAMS_PER_SUPER):
            s = sup * STREAMS_PER_SUPER + j
            cps.append(pltpu.async_copy(
                table_hbm.at[idx_v.at[s]],
                buf.at[pl.ds(j * STREAM, STREAM)],
                sem))
        return cps

    pending = [issue(0), None]
    for sup in range(n_super):
        nxt = sup + 1
        if nxt < n_super:
            pending[nxt % 2] = issue(nxt)
        for cp in pending[sup % 2]:
            cp.wait()
        pltpu.sync_copy(bufs[sup % 2],
                        out_hbm.at[pl.ds(base + sup * SUPER, SUPER)])


def _sc_gather(table4, idx4):
    n = idx4.shape[0]
    d = table4.shape[1]
    per_w = n // NUM_WORKERS
    n_super = per_w // SUPER
    assert per_w % SUPER == 0
    mesh = plsc.VectorSubcoreMesh(core_axis_name="c", subcore_axis_name="s")
    f = pl.kernel(
        functools.partial(_gather_body, per_w, n_super, d),
        out_type=jax.ShapeDtypeStruct((n, d), jnp.float32),
        mesh=mesh,
        scratch_types=[
            pltpu.VMEM((per_w // STREAM, STREAM), jnp.int32),
            pltpu.VMEM((SUPER, d), jnp.float32),
            pltpu.VMEM((SUPER, d), jnp.float32),
            pltpu.SemaphoreType.DMA,
            pltpu.SemaphoreType.DMA,
        ],
    )
    return f(table4, idx4.reshape(NUM_WORKERS, per_w // STREAM, STREAM))


def _select_matmul_body(g_ref, x_ref, w_ref, b_ref, out_ref):
    g = g_ref[0]
    xv = x_ref[0]
    pos = jnp.reshape(xv & 3, (xv.shape[1], 1))
    quarter = lax.broadcasted_iota(jnp.int32, g.shape, 1) >> 5
    masked = jnp.where(quarter == pos, g, 0.0)
    m = jnp.dot(masked, w_ref[...], preferred_element_type=jnp.float32)
    out_ref[0] = jnp.transpose(m + b_ref[...])


def _tc_select_matmul(g3, xT3, Wstack, b):
    seq, bsz, d4 = g3.shape
    o = Wstack.shape[1]
    return pl.pallas_call(
        _select_matmul_body,
        grid=(seq,),
        in_specs=[
            pl.BlockSpec((1, bsz, d4), lambda i: (i, 0, 0)),
            pl.BlockSpec((1, 1, bsz), lambda i: (i, 0, 0)),
            pl.BlockSpec((d4, o), lambda i: (0, 0)),
            pl.BlockSpec((1, o), lambda i: (0, 0)),
        ],
        out_specs=pl.BlockSpec((1, o, bsz), lambda i: (i, 0, 0)),
        out_shape=jax.ShapeDtypeStruct((seq, o, bsz), jnp.float32),
    )(g3, xT3, Wstack, b.reshape(1, o))


def kernel(x, table, W, b):
    bsz, seq = x.shape
    o = W.shape[1]
    xT = x.T.astype(jnp.int32)
    xf = xT.reshape(-1)
    table4 = table.reshape(table.shape[0] // 4, 4 * table.shape[1])
    g = _sc_gather(table4, xf >> 2)
    g3 = g.reshape(seq, bsz, table4.shape[1])
    Wstack = jnp.concatenate([W, W, W, W], axis=0)
    outT = _tc_select_matmul(g3, xT.reshape(seq, 1, bsz), Wstack, b)
    return outT.transpose(2, 0, 1)
